# Initial kernel scaffold; baseline (speedup 1.0000x reference)
import jax
import jax.numpy as jnp
from jax.experimental import pallas as pl


def kernel(h, edge_index, W1, b1, W2, b2):
    raise NotImplementedError("write your pallas kernel here")


# speedup vs baseline: 4.2319x; 4.2319x over previous
import functools

import jax
import jax.numpy as jnp
from jax import lax
from jax.experimental import pallas as pl
from jax.experimental.pallas import tpu as pltpu
from jax.experimental.pallas import tpu_sc as plsc

N = 10000
E = 160000
IN_DIM = 128
OUT_DIM = 64

NC = 2
NS = 16
NW = NC * NS
CHUNK = 128
NCHUNKS = E // CHUNK
N_PAD = 10240
ROWS_PER_TILE = N_PAD // NS


def _worker_id():
    return lax.axis_index("s") * NC + lax.axis_index("c")


def _edge_loop(wid, body):
    nk = jnp.where(wid < (NCHUNKS % NW), NCHUNKS // NW + 1, NCHUNKS // NW)

    def step(k, carry):
        body(k * NW + wid)
        return carry

    lax.fori_loop(0, nk, step, 0)


def _deg_body(src_hbm, dst_hbm, zeros_hbm, out_hbm, acc_o, acc_i, stage,
              idx_v, ones_v):
    cid = lax.axis_index("c")
    sid = lax.axis_index("s")
    wid = _worker_id()
    row0 = sid * ROWS_PER_TILE

    for j in range(CHUNK // 16):
        ones_v[pl.ds(j * 16, 16)] = jnp.ones((16,), jnp.float32)

    pltpu.sync_copy(zeros_hbm, stage)
    pltpu.sync_copy(stage, acc_o.at[pl.ds(row0, ROWS_PER_TILE)])
    pltpu.sync_copy(stage, acc_i.at[pl.ds(row0, ROWS_PER_TILE)])
    plsc.subcore_barrier()

    def body(c):
        pltpu.sync_copy(src_hbm.at[pl.ds(c * CHUNK, CHUNK)], idx_v)
        pltpu.sync_copy(ones_v, acc_o.at[idx_v], add=True)
        pltpu.sync_copy(dst_hbm.at[pl.ds(c * CHUNK, CHUNK)], idx_v)
        pltpu.sync_copy(ones_v, acc_i.at[idx_v], add=True)

    _edge_loop(wid, body)
    plsc.subcore_barrier()

    pltpu.sync_copy(acc_o.at[pl.ds(row0, ROWS_PER_TILE)], stage)
    pltpu.sync_copy(stage, out_hbm.at[cid, 0, pl.ds(row0, ROWS_PER_TILE)])
    pltpu.sync_copy(acc_i.at[pl.ds(row0, ROWS_PER_TILE)], stage)
    pltpu.sync_copy(stage, out_hbm.at[cid, 1, pl.ds(row0, ROWS_PER_TILE)])


@functools.cache
def _sc_degrees_kernel():
    return pl.kernel(
        _deg_body,
        out_type=jax.ShapeDtypeStruct((NC, 2, N_PAD), jnp.float32),
        mesh=plsc.VectorSubcoreMesh(core_axis_name="c", subcore_axis_name="s",
                                    num_cores=NC, num_subcores=NS),
        scratch_types=[
            pltpu.VMEM_SHARED((N_PAD,), jnp.float32),
            pltpu.VMEM_SHARED((N_PAD,), jnp.float32),
            pltpu.VMEM((ROWS_PER_TILE,), jnp.float32),
            pltpu.VMEM((CHUNK,), jnp.int32),
            pltpu.VMEM((CHUNK,), jnp.float32),
        ],
    )


def _segsum_body(x_hbm, src_hbm, dst_hbm, zeros_hbm, out_hbm, acc, stage,
                 idx_s, idx_d, rows_v, sem):
    cid = lax.axis_index("c")
    sid = lax.axis_index("s")
    wid = _worker_id()
    row0 = sid * ROWS_PER_TILE

    pltpu.sync_copy(zeros_hbm, stage)
    pltpu.sync_copy(stage, acc.at[pl.ds(row0, ROWS_PER_TILE)])
    plsc.subcore_barrier()

    def body(c):
        pltpu.sync_copy(src_hbm.at[pl.ds(c * CHUNK, CHUNK)], idx_s)
        pltpu.sync_copy(dst_hbm.at[pl.ds(c * CHUNK, CHUNK)], idx_d)
        pltpu.async_copy(x_hbm.at[idx_s], rows_v, sem).wait()
        pltpu.sync_copy(rows_v, acc.at[idx_d], add=True)

    _edge_loop(wid, body)
    plsc.subcore_barrier()

    pltpu.sync_copy(acc.at[pl.ds(row0, ROWS_PER_TILE)], stage)
    pltpu.sync_copy(stage, out_hbm.at[cid, pl.ds(row0, ROWS_PER_TILE)])


@functools.cache
def _sc_segsum_kernel():
    return pl.kernel(
        _segsum_body,
        out_type=jax.ShapeDtypeStruct((NC, N_PAD, OUT_DIM), jnp.float32),
        mesh=plsc.VectorSubcoreMesh(core_axis_name="c", subcore_axis_name="s",
                                    num_cores=NC, num_subcores=NS),
        scratch_types=[
            pltpu.VMEM_SHARED((N_PAD, OUT_DIM), jnp.float32),
            pltpu.VMEM((ROWS_PER_TILE, OUT_DIM), jnp.float32),
            pltpu.VMEM((CHUNK,), jnp.int32),
            pltpu.VMEM((CHUNK,), jnp.int32),
            pltpu.VMEM((CHUNK, OUT_DIM), jnp.float32),
            pltpu.SemaphoreType.DMA,
        ],
        compiler_params=pltpu.CompilerParams(use_tc_tiling_on_sc=False),
    )


def _norms(d):
    od = d[0, 0] + d[1, 0]
    idg = d[0, 1] + d[1, 1]
    ns = jnp.where(od > 0, lax.rsqrt(jnp.maximum(od, 1.0)), 0.0)
    nd = jnp.where(idg > 0, lax.rsqrt(jnp.maximum(idg, 1.0)), 0.0)
    return ns, nd


def _prep_body(h_ref, w1_ref, deg_ref, x1_ref):
    ns, _ = _norms(deg_ref[...])
    hs = h_ref[...] * ns[:N]
    x1_ref[...] = jnp.dot(hs, w1_ref[...], preferred_element_type=jnp.float32)


_tc_prep = pl.pallas_call(
    _prep_body,
    out_shape=jax.ShapeDtypeStruct((N, OUT_DIM), jnp.float32),
)


def _z_body(p_ref, deg_ref, b1_ref, z_ref, zn_ref):
    ns, nd = _norms(deg_ref[...])
    p = p_ref[...]
    s = (p[0, :N] + p[1, :N]) * nd[:N] + b1_ref[...]
    z = jax.nn.sigmoid(s)
    z_ref[...] = z
    zn_ref[...] = z * ns[:N]


_tc_z = pl.pallas_call(
    _z_body,
    out_shape=(
        jax.ShapeDtypeStruct((N, OUT_DIM), jnp.float32),
        jax.ShapeDtypeStruct((N, OUT_DIM), jnp.float32),
    ),
)


def _xhat_body(q_ref, deg_ref, w2_ref, b2_ref, xh_ref):
    _, nd = _norms(deg_ref[...])
    q = q_ref[...]
    a = (q[0, :N] + q[1, :N]) * nd[:N]
    s = jnp.dot(a, w2_ref[...], preferred_element_type=jnp.float32) + b2_ref[...]
    xh_ref[...] = jax.nn.sigmoid(s)


_tc_xhat = pl.pallas_call(
    _xhat_body,
    out_shape=jax.ShapeDtypeStruct((N, IN_DIM), jnp.float32),
)


BR = 512
BC = 2048
_GR = pl.cdiv(N, BR)
_GC = pl.cdiv(N, BC)


def _struct_body(zr_ref, zc_ref, out_ref):
    s = lax.dot_general(zr_ref[...], zc_ref[...],
                        (((1,), (1,)), ((), ())),
                        preferred_element_type=jnp.float32)
    out_ref[...] = jax.nn.sigmoid(s)


_tc_struct = pl.pallas_call(
    _struct_body,
    grid=(_GC, _GR),
    in_specs=[
        pl.BlockSpec((BR, OUT_DIM), lambda j, i: (i, 0)),
        pl.BlockSpec((BC, OUT_DIM), lambda j, i: (j, 0)),
    ],
    out_specs=pl.BlockSpec((BR, BC), lambda j, i: (i, j)),
    out_shape=jax.ShapeDtypeStruct((N, N), jnp.float32),
)


def kernel(h, edge_index, W1, b1, W2, b2):
    src = edge_index[0]
    dst = edge_index[1]
    zeros_row = jnp.zeros((ROWS_PER_TILE, OUT_DIM), jnp.float32)
    zeros_1d = jnp.zeros((ROWS_PER_TILE,), jnp.float32)

    degs = _sc_degrees_kernel()(src, dst, zeros_1d)
    degs4 = degs.reshape(NC, 2, N_PAD, 1)

    x1 = _tc_prep(h, W1, degs4)
    p = _sc_segsum_kernel()(x1, src, dst, zeros_row)
    z, zn = _tc_z(p, degs4, b1[None, :])
    q = _sc_segsum_kernel()(zn, src, dst, zeros_row)
    x_hat = _tc_xhat(q, degs4, W2, b2[None, :])
    struct = _tc_struct(z, z)
    return (struct, x_hat)
